# Initial kernel scaffold; baseline (speedup 1.0000x reference)
import jax
import jax.numpy as jnp
from jax.experimental import pallas as pl


def kernel(input, edge_index, edge_weight, W):
    raise NotImplementedError("write your pallas kernel here")


# speedup vs baseline: 4.4404x; 4.4404x over previous
import functools

import jax
import jax.numpy as jnp
from jax import lax
from jax.experimental import pallas as pl
from jax.experimental.pallas import tpu as pltpu
from jax.experimental.pallas import tpu_sc as plsc

NC = 2
NS = 16
L = 16
NW = NC * NS

CHUNK = 80
GROUPS = CHUNK // L


def _spmm_partials(x, src, dst, w):
    n, d = x.shape
    e = src.shape[0]
    epw = e // NW
    n_chunks = epw // CHUNK
    rpt = n // NS

    mesh = plsc.VectorSubcoreMesh(core_axis_name="c", subcore_axis_name="s")

    @functools.partial(
        pl.kernel,
        out_type=jax.ShapeDtypeStruct((NC, n, d), jnp.float32),
        mesh=mesh,
        scratch_types=[
            pltpu.VMEM((CHUNK,), jnp.int32),
            pltpu.VMEM((CHUNK,), jnp.int32),
            pltpu.VMEM((CHUNK,), jnp.float32),
            pltpu.VMEM((CHUNK, d), jnp.float32),
            pltpu.VMEM_SHARED((n, d), jnp.float32),
            pltpu.SemaphoreType.DMA,
        ],
    )
    def spmm(x_hbm, src_hbm, dst_hbm, w_hbm, out_hbm, sidx, didx, wts,
             rows, acc, sem):
        c = lax.axis_index("c")
        s = lax.axis_index("s")
        wid = c * NS + s

        zero = jnp.zeros((d,), jnp.float32)

        def zrow(i, _):
            rows[i, :] = zero
            return 0

        lax.fori_loop(0, CHUNK, zrow, 0)
        nz, rem = divmod(rpt, CHUNK)
        for i in range(nz):
            pltpu.sync_copy(rows, acc.at[pl.ds(s * rpt + i * CHUNK, CHUNK)])
        if rem:
            pltpu.sync_copy(rows.at[pl.ds(0, rem)],
                            acc.at[pl.ds(s * rpt + nz * CHUNK, rem)])
        plsc.subcore_barrier()

        def body(i, _):
            base = wid * epw + i * CHUNK
            pltpu.sync_copy(src_hbm.at[pl.ds(base, CHUNK)], sidx)
            pltpu.sync_copy(dst_hbm.at[pl.ds(base, CHUNK)], didx)
            pltpu.sync_copy(w_hbm.at[pl.ds(base, CHUNK)], wts)
            pltpu.async_copy(x_hbm.at[sidx], rows, sem).wait()

            def scale(g, _):
                wv = wts[pl.ds(g * L, L)]
                for j in range(L):
                    r = g * L + j
                    rows[r, :] = rows[r, :] * wv[j]
                return 0

            lax.fori_loop(0, GROUPS, scale, 0)
            pltpu.sync_copy(rows, acc.at[didx], add=True)
            return 0

        lax.fori_loop(0, n_chunks, body, 0)
        plsc.subcore_barrier()

        pltpu.sync_copy(acc.at[pl.ds(s * rpt, rpt)],
                        out_hbm.at[c, pl.ds(s * rpt, rpt)])

    return spmm(x, src, dst, w)


def _combine_matmul(partials, W, n_out):
    _, n, d = partials.shape
    bn = n // 16
    assert n % bn == 0 and bn % 8 == 0

    def body(p_ref, w_ref, o_ref):
        acc = p_ref[0] + p_ref[1]
        o_ref[...] = jnp.dot(acc, w_ref[...],
                             preferred_element_type=jnp.float32)

    out = pl.pallas_call(
        body,
        grid=(n // bn,),
        in_specs=[
            pl.BlockSpec((2, bn, d), lambda i: (0, i, 0)),
            pl.BlockSpec((d, d), lambda i: (0, 0)),
        ],
        out_specs=pl.BlockSpec((bn, d), lambda i: (i, 0)),
        out_shape=jax.ShapeDtypeStruct((n, d), jnp.float32),
    )(partials, W)
    return out[:n_out]


def kernel(input, edge_index, edge_weight, W):
    x = input
    n, d = x.shape
    src = edge_index[1]
    dst = edge_index[0]
    e = src.shape[0]

    unit = NW * CHUNK
    e_pad = ((e + unit - 1) // unit) * unit
    if e_pad != e:
        pad = e_pad - e
        src = jnp.concatenate([src, jnp.zeros((pad,), jnp.int32)])
        dst = jnp.concatenate([dst, jnp.zeros((pad,), jnp.int32)])
        edge_weight = jnp.concatenate(
            [edge_weight, jnp.zeros((pad,), jnp.float32)])

    runit = NS * 8
    n_pad = ((n + runit - 1) // runit) * runit
    if n_pad != n:
        x = jnp.pad(x, ((0, n_pad - n), (0, 0)))

    partials = _spmm_partials(x, src, dst, edge_weight)
    return _combine_matmul(partials, W, n)
